# baseline (speedup 1.0000x reference)
import functools

import jax
import jax.numpy as jnp
from jax import lax
from jax.experimental import pallas as pl
from jax.experimental.pallas import tpu as pltpu
from jax.experimental.pallas import tpu_sc as plsc

D_MODEL = 64
ROWS = 128


@functools.lru_cache(maxsize=None)
def _make_gather(vocab: int, n_j: int, n_i: int):
    info = plsc.get_sparse_core_info()
    num_workers = info.num_cores * info.num_subcores
    assert n_i == num_workers * ROWS and n_j % 2 == 0 and n_j >= 4

    mesh = plsc.VectorSubcoreMesh(core_axis_name="c", subcore_axis_name="s")

    @functools.partial(
        pl.kernel,
        mesh=mesh,
        compiler_params=pltpu.CompilerParams(needs_layout_passes=False),
        out_type=jax.ShapeDtypeStruct((n_j, D_MODEL, n_i), jnp.float32),
        scratch_types=[
            pltpu.VMEM((n_j, ROWS), jnp.int32),
            pltpu.VMEM((ROWS, D_MODEL), jnp.float32),
            pltpu.VMEM((ROWS, D_MODEL), jnp.float32),
            pltpu.VMEM((D_MODEL, ROWS), jnp.float32),
            pltpu.VMEM((D_MODEL, ROWS), jnp.float32),
            pltpu.SemaphoreType.DMA,
            pltpu.SemaphoreType.DMA,
            pltpu.SemaphoreType.DMA,
            pltpu.SemaphoreType.DMA,
        ],
    )
    def gather_kernel(idx_hbm, table_hbm, out_hbm, idx_v,
                      gr0, gr1, tr0, tr1, gs0, gs1, ws0, ws1):
        wid = lax.axis_index("s") * info.num_cores + lax.axis_index("c")
        base = wid * ROWS
        grow = (gr0, gr1)
        tr = (tr0, tr1)
        gsem = (gs0, gs1)
        wsem = (ws0, ws1)

        pltpu.sync_copy(idx_hbm.at[:, pl.ds(base, ROWS)], idx_v)

        iota = lax.iota(jnp.int32, 16)
        cvec = tuple(iota + 16 * k for k in range(4))

        def merged(j, bt, fire):
            bn = 1 - bt

            def grp(g, carry):
                if fire:
                    ivec = idx_v.at[j + 1][pl.ds(g * 16, 16)]
                for l in range(16):
                    i = g * 16 + l
                    if fire:
                        pltpu.async_copy(
                            table_hbm.at[pl.ds(ivec[l], 1)],
                            grow[bn].at[pl.ds(i, 1)],
                            gsem[bn],
                        )
                    row = grow[bt].at[i]
                    isplat = jnp.full((16,), 0, jnp.int32) + i
                    for k in range(4):
                        vec = row[pl.ds(16 * k, 16)]
                        plsc.store_scatter(tr[bt], [cvec[k], isplat], vec)
                return carry

            lax.fori_loop(0, ROWS // 16, grp, 0)

        def wait_gather(b):
            pltpu.make_async_copy(
                table_hbm.at[pl.ds(0, ROWS)], grow[b], gsem[b]
            ).wait()

        def fire_wb(j, b):
            pltpu.async_copy(
                tr[b], out_hbm.at[j, :, pl.ds(base, ROWS)], wsem[b]
            )

        def wait_wb(j, b):
            pltpu.make_async_copy(
                tr[b], out_hbm.at[j, :, pl.ds(base, ROWS)], wsem[b]
            ).wait()

        def fire_gather(j, b):
            def grp(g, carry):
                ivec = idx_v.at[j][pl.ds(g * 16, 16)]
                for l in range(16):
                    pltpu.async_copy(
                        table_hbm.at[pl.ds(ivec[l], 1)],
                        grow[b].at[pl.ds(g * 16 + l, 1)],
                        gsem[b],
                    )
                return carry
            lax.fori_loop(0, ROWS // 16, grp, 0)

        fire_gather(0, 0)
        wait_gather(0)
        merged(0, 0, fire=True)
        fire_wb(0, 0)
        wait_gather(1)
        merged(1, 1, fire=True)
        fire_wb(1, 1)

        def body(k, carry):
            for m in range(2):
                t = 2 + 2 * k + m
                bt = m
                wait_gather(bt)
                wait_wb(t - 2, bt)
                merged(t, bt, fire=True)
                fire_wb(t, bt)
            return carry

        lax.fori_loop(0, (n_j - 4) // 2, body, 0)

        for t in (n_j - 2, n_j - 1):
            bt = t % 2
            wait_gather(bt)
            wait_wb(t - 2, bt)
            merged(t, bt, fire=(t == n_j - 2))
            fire_wb(t, bt)
        wait_wb(n_j - 2, (n_j - 2) % 2)
        wait_wb(n_j - 1, (n_j - 1) % 2)

    return gather_kernel


def kernel(x, table):
    n_i, n_j = x.shape
    xt = jnp.transpose(x, (1, 0)).astype(jnp.int32)
    out = _make_gather(table.shape[0], n_j, n_i)(xt, table)
    return jnp.transpose(out, (2, 0, 1))


# speedup vs baseline: 1.6011x; 1.6011x over previous
import functools

import jax
import jax.numpy as jnp
from jax import lax
from jax.experimental import pallas as pl
from jax.experimental.pallas import tpu as pltpu
from jax.experimental.pallas import tpu_sc as plsc

D_MODEL = 64
NBUF = 4


@functools.lru_cache(maxsize=None)
def _make_gather(vocab: int, n_j: int, n_i: int):
    info = plsc.get_sparse_core_info()
    num_workers = info.num_cores * info.num_subcores
    rows = n_i // num_workers
    assert n_i % num_workers == 0 and rows <= 128 and n_j % NBUF == 0

    mesh = plsc.VectorSubcoreMesh(core_axis_name="c", subcore_axis_name="s")

    @functools.partial(
        pl.kernel,
        mesh=mesh,
        out_type=jax.ShapeDtypeStruct((n_i, n_j, D_MODEL), jnp.float32),
        scratch_types=[
            pltpu.VMEM((n_j, rows), jnp.int32),
        ]
        + [pltpu.VMEM((rows, D_MODEL), jnp.float32)] * NBUF
        + [pltpu.SemaphoreType.DMA] * (2 * NBUF),
    )
    def gather_kernel(idx_hbm, table_hbm, out_hbm, idx_v,
                      g0, g1, g2, g3, gs0, gs1, gs2, gs3, ws0, ws1, ws2, ws3):
        wid = lax.axis_index("s") * info.num_cores + lax.axis_index("c")
        base = wid * rows
        grow = (g0, g1, g2, g3)
        gsem = (gs0, gs1, gs2, gs3)
        wsem = (ws0, ws1, ws2, ws3)

        pltpu.sync_copy(idx_hbm.at[:, pl.ds(base, rows)], idx_v)

        def fire_gather(j, b):
            def grp_body(g, carry):
                vec = idx_v[j, pl.ds(g * 16, 16)]
                for l in range(16):
                    pltpu.async_copy(
                        table_hbm.at[pl.ds(vec[l], 1)],
                        grow[b].at[pl.ds(g * 16 + l, 1)],
                        gsem[b],
                    )
                return carry
            lax.fori_loop(0, rows // 16, grp_body, 0)

        def wait_gather(b):
            pltpu.make_async_copy(
                table_hbm.at[pl.ds(0, rows)], grow[b], gsem[b]
            ).wait()

        def fire_wb(j, b):
            pltpu.async_copy(grow[b], out_hbm.at[pl.ds(base, rows), j], wsem[b])

        def wait_wb(j, b):
            pltpu.make_async_copy(
                grow[b], out_hbm.at[pl.ds(base, rows), j], wsem[b]
            ).wait()

        for j in range(NBUF - 1):
            fire_gather(j, j)

        wait_gather(0)
        fire_wb(0, 0)
        fire_gather(NBUF - 1, NBUF - 1)

        def body(k, carry):
            for m in range(NBUF):
                j = NBUF * k + 1 + m
                b = (1 + m) % NBUF
                wait_gather(b)
                fire_wb(j, b)
                wait_wb(j - 1, (b - 1) % NBUF)
                fire_gather(j + NBUF - 1, (b + NBUF - 1) % NBUF)
            return carry

        lax.fori_loop(0, (n_j - NBUF) // NBUF, body, 0)

        for m in range(NBUF - 1):
            j = n_j - (NBUF - 1) + m
            b = j % NBUF
            wait_gather(b)
            fire_wb(j, b)
        for m in range(NBUF):
            j = n_j - NBUF + m
            wait_wb(j, j % NBUF)

    return gather_kernel


def kernel(x, table):
    n_i, n_j = x.shape
    xt = jnp.transpose(x, (1, 0)).astype(jnp.int32)
    return _make_gather(table.shape[0], n_j, n_i)(xt, table)
